# baseline (speedup 1.0000x reference)
import jax
import jax.numpy as jnp
from jax.experimental import pallas as pl
from jax.experimental.pallas import tpu as pltpu

N = 256
H = 128
C = 768
P = 49
D = C * P
PBLK = 7
NB = P // PBLK


def _mask_loss_kernel(x_ref, m_ref, out_ref, s_acc, rs_acc, stash):
    i = pl.program_id(0)
    j = pl.program_id(1)

    @pl.when((i == 0) & (j == 0))
    def _init():
        s_acc[:, :] = jnp.zeros_like(s_acc)
        rs_acc[:, :] = jnp.zeros_like(rs_acc)

    dims = (((1,), (1,)), ((), ()))

    @pl.when(j == 0)
    def _low_half():
        s11 = s_acc[0:H, 0:H]
        rs = rs_acc[0:H, :]
        for t in range(PBLK):
            a = x_ref[t, 0, :, :]
            stash[t, :, :] = a
            s11 = s11 + jax.lax.dot_general(
                a, a, dims, preferred_element_type=jnp.float32
            )
            rs = rs + jnp.sum(a, axis=1, keepdims=True)
        s_acc[0:H, 0:H] = s11
        rs_acc[0:H, :] = rs

    @pl.when(j == 1)
    def _high_half():
        s22 = s_acc[H:N, H:N]
        s12 = s_acc[0:H, H:N]
        rs = rs_acc[H:N, :]
        for t in range(PBLK):
            b = x_ref[t, 0, :, :]
            lo = stash[t, :, :]
            s22 = s22 + jax.lax.dot_general(
                b, b, dims, preferred_element_type=jnp.float32
            )
            s12 = s12 + jax.lax.dot_general(
                lo, b, dims, preferred_element_type=jnp.float32
            )
            rs = rs + jnp.sum(b, axis=1, keepdims=True)
        s_acc[H:N, H:N] = s22
        s_acc[0:H, H:N] = s12
        rs_acc[H:N, :] = rs

    @pl.when((i == NB - 1) & (j == 1))
    def _finalize():
        s_acc[H:N, 0:H] = s_acc[0:H, H:N].T
        total = rs_acc[:, 0:1]
        g = s_acc[:, :] - (total * total.T) * (1.0 / D)
        rows = jax.lax.broadcasted_iota(jnp.int32, (N, N), 0)
        cols = jax.lax.broadcasted_iota(jnp.int32, (N, N), 1)
        diag = jnp.sum(jnp.where(rows == cols, g, 0.0), axis=1, keepdims=True)
        inv = jax.lax.rsqrt(diag)
        corr = g * inv * inv.T
        mr = m_ref[:, :]
        mc = mr.T
        act = mr > 0.0
        masked = jnp.where(act.T & act, corr, 0.0) * (mc * mr)
        out_ref[:, :] = jnp.sqrt(jnp.sum(masked * masked, keepdims=True))


def kernel(weights, mask):
    x = weights.transpose(2, 3, 0, 1).reshape(P, 2, H, C)
    m = mask.reshape(1, N)
    out = pl.pallas_call(
        _mask_loss_kernel,
        grid=(NB, 2),
        in_specs=[
            pl.BlockSpec((PBLK, 1, H, C), lambda i, j: (i, j, 0, 0)),
            pl.BlockSpec((1, N), lambda i, j: (0, 0)),
        ],
        out_specs=pl.BlockSpec((1, 1), lambda i, j: (0, 0)),
        out_shape=jax.ShapeDtypeStruct((1, 1), jnp.float32),
        scratch_shapes=[
            pltpu.VMEM((N, N), jnp.float32),
            pltpu.VMEM((N, 1), jnp.float32),
            pltpu.VMEM((PBLK, H, C), jnp.float32),
        ],
    )(x, m)
    return out[0, 0]
